# baseline (speedup 1.0000x reference)
import functools

import jax
import jax.numpy as jnp
from jax.experimental import pallas as pl
from jax.experimental.pallas import tpu as pltpu

_BM = 512


def _mm_kernel(x_ref, w_ref, b_ref, o_ref):
    acc = jnp.dot(x_ref[...], w_ref[...], preferred_element_type=jnp.float32)
    o_ref[...] = jnp.maximum(acc + b_ref[...], 0.0)


@functools.partial(jax.jit, static_argnames=())
def kernel(flat_values, cu_seqlens, W, b):
    del cu_seqlens
    M, K = flat_values.shape
    N = W.shape[1]
    b2 = b.reshape(1, N)
    grid = (M // _BM,)
    out = pl.pallas_call(
        _mm_kernel,
        grid=grid,
        in_specs=[
            pl.BlockSpec((_BM, K), lambda i: (i, 0)),
            pl.BlockSpec((K, N), lambda i: (0, 0)),
            pl.BlockSpec((1, N), lambda i: (0, 0)),
        ],
        out_specs=pl.BlockSpec((_BM, N), lambda i: (i, 0)),
        out_shape=jax.ShapeDtypeStruct((M, N), jnp.float32),
        compiler_params=pltpu.CompilerParams(
            dimension_semantics=("arbitrary",),
        ),
    )(flat_values, W, b2)
    return out


# speedup vs baseline: 1.2485x; 1.2485x over previous
import functools

import jax
import jax.numpy as jnp
from jax.experimental import pallas as pl
from jax.experimental.pallas import tpu as pltpu

_BM = 2048


def _mm_kernel(x_ref, w_ref, b_ref, o_ref):
    acc = jnp.dot(x_ref[...], w_ref[...], preferred_element_type=jnp.float32)
    o_ref[...] = jnp.maximum(acc + b_ref[...], 0.0)


@functools.partial(jax.jit, static_argnames=())
def kernel(flat_values, cu_seqlens, W, b):
    del cu_seqlens
    M, K = flat_values.shape
    N = W.shape[1]
    b2 = b.reshape(1, N)
    grid = (M // _BM,)
    out = pl.pallas_call(
        _mm_kernel,
        grid=grid,
        in_specs=[
            pl.BlockSpec((_BM, K), lambda i: (i, 0)),
            pl.BlockSpec((K, N), lambda i: (0, 0)),
            pl.BlockSpec((1, N), lambda i: (0, 0)),
        ],
        out_specs=pl.BlockSpec((_BM, N), lambda i: (i, 0)),
        out_shape=jax.ShapeDtypeStruct((M, N), jnp.float32),
        compiler_params=pltpu.CompilerParams(
            dimension_semantics=("arbitrary",),
        ),
    )(flat_values, W, b2)
    return out


# speedup vs baseline: 1.4224x; 1.1392x over previous
import functools

import jax
import jax.numpy as jnp
from jax.experimental import pallas as pl
from jax.experimental.pallas import tpu as pltpu

_BM = 1024
_D = 4
_OD = 4
_NH = 512


def _mm_kernel(x_hbm, w_ref, b_ref, o_hbm, x_vmem, o_vmem, x_sems, o_sems):
    i = pl.program_id(0)
    s_total = pl.num_programs(0)

    def x_copy(step, slot):
        return pltpu.make_async_copy(
            x_hbm.at[pl.ds(step * _BM, _BM), :], x_vmem.at[slot],
            x_sems.at[slot])

    def o_copy(step, slot, h):
        cols = pl.ds(h * _NH, _NH)
        return pltpu.make_async_copy(
            o_vmem.at[slot, :, cols],
            o_hbm.at[pl.ds(step * _BM, _BM), cols],
            o_sems.at[slot])

    @pl.when(i == 0)
    def _():
        for j in range(_D):
            x_copy(j, j).start()

    slot = jax.lax.rem(i, _D)
    oslot = jax.lax.rem(i, _OD)

    @pl.when(i >= _OD)
    def _():
        o_copy(i - _OD, oslot, 0).wait()
        o_copy(i - _OD, oslot, 1).wait()

    x_copy(i, slot).wait()
    x = x_vmem[slot]
    for h in range(2):
        cols = pl.ds(h * _NH, _NH)
        acc = jnp.dot(x, w_ref[:, cols], preferred_element_type=jnp.float32)
        o_vmem[oslot, :, cols] = jnp.maximum(acc + b_ref[:, cols], 0.0)
        o_copy(i, oslot, h).start()

    @pl.when(i + _D < s_total)
    def _():
        x_copy(i + _D, slot).start()

    @pl.when(i == s_total - 1)
    def _():
        for j in range(_OD):
            step = s_total - _OD + j
            sl = jax.lax.rem(jnp.int32(step), _OD)
            o_copy(step, sl, 0).wait()
            o_copy(step, sl, 1).wait()


@functools.partial(jax.jit, static_argnames=())
def kernel(flat_values, cu_seqlens, W, b):
    del cu_seqlens
    M, K = flat_values.shape
    N = W.shape[1]
    b2 = b.reshape(1, N)
    grid = (M // _BM,)
    out = pl.pallas_call(
        _mm_kernel,
        grid=grid,
        in_specs=[
            pl.BlockSpec(memory_space=pltpu.MemorySpace.HBM),
            pl.BlockSpec((K, N), lambda i: (0, 0)),
            pl.BlockSpec((1, N), lambda i: (0, 0)),
        ],
        out_specs=pl.BlockSpec(memory_space=pltpu.MemorySpace.HBM),
        out_shape=jax.ShapeDtypeStruct((M, N), jnp.float32),
        scratch_shapes=[
            pltpu.VMEM((_D, _BM, K), jnp.float32),
            pltpu.VMEM((_OD, _BM, N), jnp.float32),
            pltpu.SemaphoreType.DMA((_D,)),
            pltpu.SemaphoreType.DMA((_OD,)),
        ],
        compiler_params=pltpu.CompilerParams(
            dimension_semantics=("arbitrary",),
        ),
    )(flat_values, W, b2)
    return out
